# Initial kernel scaffold; baseline (speedup 1.0000x reference)
import jax
import jax.numpy as jnp
from jax.experimental import pallas as pl


def kernel(ans_emb, ocr_emb, labels, tt_table, ans_ln_w, ans_ln_b, ocr_ln_w, ocr_ln_b, emb_ln_w, emb_ln_b):
    raise NotImplementedError("write your pallas kernel here")


# speedup vs baseline: 5.1149x; 5.1149x over previous
import functools
import math

import jax
import jax.numpy as jnp
import numpy as np
from jax import lax
from jax.experimental import pallas as pl
from jax.experimental.pallas import tpu as pltpu
from jax.experimental.pallas import tpu_sc as plsc

HIDDEN = 256
ANS_NUM = 5000
OCR_NUM = 100
BATCH = 128
SEQ = 128
LN_EPS = 1e-12

ROWS_OCR = BATCH * OCR_NUM
EMB_BASE = ANS_NUM + ROWS_OCR
TAB_ROWS = 18200
BLK = 200
N_ANS_BLK = ANS_NUM // BLK
N_OCR_BLK = ROWS_OCR // BLK
N_EMB_BLK = (TAB_ROWS - EMB_BASE) // BLK
GRID = TAB_ROWS // BLK
LANES = 16


def _make_pe(d_model=HIDDEN, max_len=SEQ):
    position = np.arange(max_len, dtype=np.float64)[:, None]
    div_term = np.exp(np.arange(0, d_model, 2, dtype=np.float64) * (-math.log(10000.0) / d_model))
    pe = np.zeros((max_len, d_model), dtype=np.float32)
    pe[:, 0::2] = np.sin(position / div_term)
    pe[:, 1::2] = np.cos(position / div_term)
    return pe


_PE_PAD = np.concatenate(
    [_make_pe(), _make_pe(), np.zeros((TAB_ROWS - EMB_BASE - 2 * SEQ, HIDDEN), np.float32)], axis=0
)


def _ln(x, w, b):
    mu = jnp.mean(x, axis=1, keepdims=True)
    var = jnp.mean((x - mu) ** 2, axis=1, keepdims=True)
    return (x - mu) * lax.rsqrt(var + LN_EPS) * w + b


def _norm_body(ans_ref, ocr_ref, pe_ref, params_ref, tt_ref, out_ref):
    g = pl.program_id(0)

    @pl.when(g < N_ANS_BLK)
    def _():
        out_ref[...] = _ln(ans_ref[...], params_ref[0], params_ref[1])

    @pl.when(jnp.logical_and(g >= N_ANS_BLK, g < N_ANS_BLK + N_OCR_BLK))
    def _():
        out_ref[...] = _ln(ocr_ref[...], params_ref[2], params_ref[3])

    @pl.when(g >= N_ANS_BLK + N_OCR_BLK)
    def _():
        rid = (g - (N_ANS_BLK + N_OCR_BLK)) * BLK + lax.broadcasted_iota(jnp.int32, (BLK, 1), 0)
        tt = jnp.where(rid < SEQ, tt_ref[0], tt_ref[1])
        out_ref[...] = _ln(pe_ref[...] + tt, params_ref[4], params_ref[5])


def _normalize_table(ans_emb, ocr_flat, pe_pad, params, tt_table):
    return pl.pallas_call(
        _norm_body,
        grid=(GRID,),
        in_specs=[
            pl.BlockSpec((BLK, HIDDEN), lambda g: (jnp.minimum(g, N_ANS_BLK - 1), 0)),
            pl.BlockSpec((BLK, HIDDEN), lambda g: (jnp.clip(g - N_ANS_BLK, 0, N_OCR_BLK - 1), 0)),
            pl.BlockSpec((BLK, HIDDEN), lambda g: (jnp.clip(g - (N_ANS_BLK + N_OCR_BLK), 0, N_EMB_BLK - 1), 0)),
            pl.BlockSpec((6, HIDDEN), lambda g: (0, 0)),
            pl.BlockSpec((2, HIDDEN), lambda g: (0, 0)),
        ],
        out_specs=pl.BlockSpec((BLK, HIDDEN), lambda g: (g, 0)),
        out_shape=jax.ShapeDtypeStruct((TAB_ROWS, HIDDEN), jnp.float32),
    )(ans_emb, ocr_flat, pe_pad, params, tt_table)


def _sc_gather(table, labels_flat):
    info = plsc.get_sparse_core_info()
    nc, ns = info.num_cores, info.num_subcores
    nw = nc * ns
    b_per_w = BATCH // nw
    mesh = plsc.VectorSubcoreMesh(core_axis_name="c", subcore_axis_name="s")

    @functools.partial(
        pl.kernel,
        mesh=mesh,
        out_type=jax.ShapeDtypeStruct((BATCH * SEQ, HIDDEN), jnp.float32),
        scratch_types=[
            pltpu.VMEM((b_per_w * SEQ,), jnp.int32),
            pltpu.VMEM((SEQ,), jnp.int32),
            pltpu.VMEM((SEQ,), jnp.int32),
            pltpu.VMEM((2 * SEQ, HIDDEN), jnp.float32),
            pltpu.SemaphoreType.DMA,
        ],
    )
    def k(table_hbm, lbl_hbm, out_hbm, lbl_v, gi_v, ei_v, rows_v, sem):
        wid = lax.axis_index("s") * nc + lax.axis_index("c")
        pltpu.sync_copy(lbl_hbm.at[pl.ds(wid * (b_per_w * SEQ), b_per_w * SEQ)], lbl_v)
        for bi in range(b_per_w):
            b = wid * b_per_w + bi
            for j in range(SEQ // LANES):
                v = lbl_v[pl.ds(bi * SEQ + j * LANES, LANES)]
                is_ocr = v >= ANS_NUM
                gi_v[pl.ds(j * LANES, LANES)] = jnp.where(is_ocr, v + b * OCR_NUM, v)
                s_vec = lax.iota(jnp.int32, LANES) + (j * LANES + EMB_BASE)
                ei_v[pl.ds(j * LANES, LANES)] = jnp.where(is_ocr, s_vec + SEQ, s_vec)
            cp1 = pltpu.async_copy(table_hbm.at[gi_v], rows_v.at[pl.ds(0, SEQ)], sem)
            cp2 = pltpu.async_copy(table_hbm.at[ei_v], rows_v.at[pl.ds(SEQ, SEQ)], sem)
            cp1.wait()
            cp2.wait()

            def add_row(s, carry):
                for h in range(HIDDEN // LANES):
                    sl = pl.ds(h * LANES, LANES)
                    rows_v[s, sl] = rows_v[s, sl] + rows_v[s + SEQ, sl]
                return carry

            lax.fori_loop(0, SEQ, add_row, 0)
            pltpu.sync_copy(rows_v.at[pl.ds(0, SEQ)], out_hbm.at[pl.ds(b * SEQ, SEQ)])

    return k(table, labels_flat)


def kernel(ans_emb, ocr_emb, labels, tt_table, ans_ln_w, ans_ln_b, ocr_ln_w, ocr_ln_b, emb_ln_w, emb_ln_b):
    batch, seq = labels.shape
    ocr_flat = ocr_emb.reshape(-1, HIDDEN)
    params = jnp.stack([ans_ln_w, ans_ln_b, ocr_ln_w, ocr_ln_b, emb_ln_w, emb_ln_b])
    table = _normalize_table(ans_emb, ocr_flat, jnp.asarray(_PE_PAD), params, tt_table)
    flat = _sc_gather(table, labels.reshape(-1).astype(jnp.int32))
    return flat.reshape(batch, seq, HIDDEN)
